# baseline (device time: 13985 ns/iter reference)
import jax
import jax.numpy as jnp
from jax import lax
from jax.experimental import pallas as pl
from jax.experimental.pallas import tpu as pltpu

N_DEV = 4
N_EXP = 8
CAP = 102


def kernel(x, router_W, route_idx, expert_W):
    T, D = x.shape
    E_loc, _, H = expert_W.shape

    def body(x_ref, rw_ref, idx_ref, ew_ref, out_ref,
             w_all, hist_all, my_hist,
             w_send_sems, w_recv_sems, h_send_sems, h_recv_sems):
        my = lax.axis_index("i")

        barrier = pltpu.get_barrier_semaphore()
        for k in range(1, N_DEV):
            pl.semaphore_signal(
                barrier, inc=1,
                device_id=((my + k) % N_DEV,),
                device_id_type=pl.DeviceIdType.MESH,
            )
        pl.semaphore_wait(barrier, N_DEV - 1)

        eids = lax.broadcasted_iota(jnp.int32, (T, N_EXP), 1)
        onehot = (idx_ref[:, :] == eids).astype(jnp.float32)
        my_hist[...] = jnp.sum(onehot, axis=0, keepdims=True)

        w_all[pl.ds(my * E_loc, E_loc)] = ew_ref[...]
        hist_all[pl.ds(my, 1)] = my_hist[...][None]

        w_sends = []
        h_sends = []
        for k in range(1, N_DEV):
            p = (my + k) % N_DEV
            wr = pltpu.make_async_remote_copy(
                src_ref=ew_ref,
                dst_ref=w_all.at[pl.ds(my * E_loc, E_loc)],
                send_sem=w_send_sems.at[k - 1],
                recv_sem=w_recv_sems.at[my],
                device_id=(p,),
                device_id_type=pl.DeviceIdType.MESH,
            )
            wr.start()
            w_sends.append(wr)
            hr = pltpu.make_async_remote_copy(
                src_ref=my_hist,
                dst_ref=hist_all.at[my],
                send_sem=h_send_sems.at[k - 1],
                recv_sem=h_recv_sems.at[my],
                device_id=(p,),
                device_id_type=pl.DeviceIdType.MESH,
            )
            hr.start()
            h_sends.append(hr)

        ri = lax.broadcasted_iota(jnp.int32, (T, T), 0)
        ci = lax.broadcasted_iota(jnp.int32, (T, T), 1)
        tril = (ci < ri).astype(jnp.float32)
        excl = jnp.dot(tril, onehot, preferred_element_type=jnp.float32)

        for k in range(1, N_DEV):
            p = (my + k) % N_DEV
            pltpu.make_async_remote_copy(
                src_ref=my_hist,
                dst_ref=hist_all.at[p],
                send_sem=h_send_sems.at[k - 1],
                recv_sem=h_recv_sems.at[p],
                device_id=(p,),
                device_id_type=pl.DeviceIdType.MESH,
            ).wait_recv()
        hs = hist_all[:, 0, :]
        dmask = (lax.broadcasted_iota(jnp.int32, (N_DEV, 1), 0)
                 < my).astype(jnp.float32)
        offsets = jnp.sum(hs * dmask, axis=0, keepdims=True)

        rank = offsets + excl
        keep = onehot * (rank < CAP).astype(jnp.float32)

        for k in range(1, N_DEV):
            p = (my + k) % N_DEV
            pltpu.make_async_remote_copy(
                src_ref=ew_ref,
                dst_ref=w_all.at[pl.ds(p * E_loc, E_loc)],
                send_sem=w_send_sems.at[k - 1],
                recv_sem=w_recv_sems.at[p],
                device_id=(p,),
                device_id_type=pl.DeviceIdType.MESH,
            ).wait_recv()

        xv = x_ref[...]
        acc = jnp.zeros((T, H), jnp.float32)
        for e in range(N_EXP):
            y = jnp.dot(xv, w_all[e], preferred_element_type=jnp.float32)
            acc = acc + keep[:, e][:, None] * y
        out_ref[...] = acc

        for r in w_sends + h_sends:
            r.wait_send()

    return pl.pallas_call(
        body,
        out_shape=jax.ShapeDtypeStruct((T, H), jnp.float32),
        in_specs=[pl.BlockSpec(memory_space=pltpu.VMEM)] * 4,
        out_specs=pl.BlockSpec(memory_space=pltpu.VMEM),
        scratch_shapes=[
            pltpu.VMEM((N_EXP, D, H), jnp.float32),
            pltpu.VMEM((N_DEV, 1, N_EXP), jnp.float32),
            pltpu.VMEM((1, N_EXP), jnp.float32),
            pltpu.SemaphoreType.DMA((N_DEV - 1,)),
            pltpu.SemaphoreType.DMA((N_DEV,)),
            pltpu.SemaphoreType.DMA((N_DEV - 1,)),
            pltpu.SemaphoreType.DMA((N_DEV,)),
        ],
        compiler_params=pltpu.CompilerParams(collective_id=0),
    )(x, router_W, route_idx, expert_W)
